# baseline (speedup 1.0000x reference)
import functools

import jax
import jax.numpy as jnp
from jax import lax
from jax.experimental import pallas as pl
from jax.experimental.pallas import tpu as pltpu
from jax.experimental.pallas import tpu_sc as plsc

_EMBED_DIM = 16
_BATCH = 16384
_NUM_CORES = 2
_NUM_SUBCORES = 16
_NUM_WORKERS = _NUM_CORES * _NUM_SUBCORES
_ROWS_PER_WORKER = _BATCH // _NUM_WORKERS
_CHUNK = 128
_NUM_CHUNKS = _ROWS_PER_WORKER // _CHUNK


def _build_gather():
  mesh = plsc.VectorSubcoreMesh(core_axis_name="c", subcore_axis_name="s")

  @functools.partial(
      pl.kernel,
      mesh=mesh,
      out_type=jax.ShapeDtypeStruct((_EMBED_DIM, _BATCH), jnp.float32),
      scratch_types=[
          pltpu.VMEM((_ROWS_PER_WORKER,), jnp.int32),
          pltpu.VMEM((_EMBED_DIM, _ROWS_PER_WORKER), jnp.float32),
          pltpu.SemaphoreType.DMA,
      ],
      compiler_params=pltpu.CompilerParams(use_tc_tiling_on_sc=False),
  )
  def gather(idx_hbm, table_t_hbm, out_t_hbm, idx_v, rows_v, sem):
    wid = lax.axis_index("s") * _NUM_CORES + lax.axis_index("c")
    base = wid * _ROWS_PER_WORKER
    pltpu.sync_copy(idx_hbm.at[pl.ds(base, _ROWS_PER_WORKER)], idx_v)
    copies = []
    for d in range(_EMBED_DIM):
      for j in range(_NUM_CHUNKS):
        copies.append(
            pltpu.async_copy(
                table_t_hbm.at[d].at[idx_v.at[pl.ds(j * _CHUNK, _CHUNK)]],
                rows_v.at[d].at[pl.ds(j * _CHUNK, _CHUNK)],
                sem,
            )
        )
    for c in copies:
      c.wait()
    pltpu.sync_copy(rows_v, out_t_hbm.at[:, pl.ds(base, _ROWS_PER_WORKER)])

  return gather


_GATHER = _build_gather()


def kernel(x, table):
  out_t = _GATHER(x.astype(jnp.int32), table.T)
  return out_t.T


# speedup vs baseline: 19.5261x; 19.5261x over previous
import functools

import jax
import jax.numpy as jnp
from jax import lax
from jax.experimental import pallas as pl
from jax.experimental.pallas import tpu as pltpu
from jax.experimental.pallas import tpu_sc as plsc

_EMBED_DIM = 16
_BATCH = 16384
_NUM_CORES = 2
_NUM_SUBCORES = 16
_NUM_WORKERS = _NUM_CORES * _NUM_SUBCORES
_ROWS_PER_WORKER = _BATCH // _NUM_WORKERS
_GROUP = 16
_NUM_GROUPS = _ROWS_PER_WORKER // _GROUP
_TILE_W = 128


def _build_gather():
  mesh = plsc.VectorSubcoreMesh(core_axis_name="c", subcore_axis_name="s")

  @functools.partial(
      pl.kernel,
      mesh=mesh,
      out_type=jax.ShapeDtypeStruct((_EMBED_DIM, _BATCH), jnp.float32),
      scratch_types=[
          pltpu.VMEM((_ROWS_PER_WORKER,), jnp.int32),
          pltpu.VMEM((2, _EMBED_DIM, _GROUP * _TILE_W), jnp.float32),
          pltpu.VMEM((_EMBED_DIM, _ROWS_PER_WORKER), jnp.float32),
          pltpu.SemaphoreType.DMA,
      ],
      compiler_params=pltpu.CompilerParams(needs_layout_passes=False),
  )
  def gather(idx_hbm, table_t_hbm, out_t_hbm, idx_v, bufs, rows_v, sem):
    wid = lax.axis_index("s") * _NUM_CORES + lax.axis_index("c")
    base = wid * _ROWS_PER_WORKER
    pltpu.sync_copy(idx_hbm.at[pl.ds(base, _ROWS_PER_WORKER)], idx_v)

    row_iota = lax.iota(jnp.int32, _EMBED_DIM)

    def issue_group(g):
      b = lax.rem(g, 2)
      xv = idx_v[pl.ds(g * _GROUP, _GROUP)]
      tile_off = (xv >> 7) << 7
      for k in range(_GROUP):
        off = pl.multiple_of(tile_off[k], _TILE_W)
        pltpu.async_copy(
            table_t_hbm.at[:, pl.ds(off, _TILE_W)],
            bufs.at[b, :, pl.ds(k * _TILE_W, _TILE_W)],
            sem,
        )

    def drain_extract_group(g):
      b = lax.rem(g, 2)
      xv = idx_v[pl.ds(g * _GROUP, _GROUP)]
      lane = xv & 127
      for k in range(_GROUP):
        pltpu.make_async_copy(
            table_t_hbm.at[:, pl.ds(0, _TILE_W)],
            bufs.at[b, :, pl.ds(k * _TILE_W, _TILE_W)],
            sem,
        ).wait()
      for k in range(_GROUP):
        col = jnp.full((_EMBED_DIM,), lane[k] + k * _TILE_W, jnp.int32)
        vals = plsc.load_gather(bufs.at[b], [row_iota, col])
        out_col = jnp.full((_EMBED_DIM,), g * _GROUP + k, jnp.int32)
        plsc.store_scatter(rows_v, [row_iota, out_col], vals)

    def step(g, _):
      @pl.when(g < _NUM_GROUPS)
      def _issue():
        issue_group(g)

      @pl.when(g > 0)
      def _drain():
        drain_extract_group(g - 1)

      return _

    lax.fori_loop(0, _NUM_GROUPS + 1, step, None)
    pltpu.sync_copy(rows_v, out_t_hbm.at[:, pl.ds(base, _ROWS_PER_WORKER)])

  return gather


_GATHER = _build_gather()


def kernel(x, table):
  out_t = _GATHER(x.astype(jnp.int32), table.T)
  return out_t.T
